# baseline (speedup 1.0000x reference)
import jax
import jax.numpy as jnp
from jax.experimental import pallas as pl

_B, _P, _C = 16384, 1000, 4
_BB = 256
_K = 128


def _onehot_body(x_ref, o_ref):
    xq = x_ref[...].astype(jnp.bfloat16)
    ei = jax.lax.broadcasted_iota(jnp.int32, (_K, _K * _C), 0)
    ej = jax.lax.broadcasted_iota(jnp.int32, (_K, _K * _C), 1)
    e = (jax.lax.shift_right_logical(ej, 2) == ei).astype(jnp.bfloat16)
    cj = jax.lax.broadcasted_iota(jnp.int32, (1, _K * _C), 1)
    cmp = ((cj & (_C - 1)) + 1).astype(jnp.float32)
    one = jnp.float32(1.0)
    zero = jnp.float32(0.0)
    nchunks = (_P + _K - 1) // _K
    for k in range(nchunks):
        off = _K * k
        w = min(_K, _P - off)
        xs = xq[:, off:off + w]
        ek = e if w == _K else e[:w, :w * _C]
        xr = jax.lax.dot_general(
            xs, ek, (((1,), (0,)), ((), ())),
            preferred_element_type=jnp.float32,
        )
        o_ref[:, _C * off:_C * (off + w)] = jnp.where(
            xr == cmp[:, :w * _C], one, zero)


def kernel(x):
    out2 = pl.pallas_call(
        _onehot_body,
        grid=(_B // _BB,),
        in_specs=[pl.BlockSpec((_BB, _P), lambda i: (i, 0))],
        out_specs=pl.BlockSpec((_BB, _P * _C), lambda i: (i, 0)),
        out_shape=jax.ShapeDtypeStruct((_B, _P * _C), jnp.float32),
    )(x)
    return out2.reshape(_B, _P, _C)


# speedup vs baseline: 6.1038x; 6.1038x over previous
import jax
import jax.numpy as jnp
from jax.experimental import pallas as pl

_B, _P, _C = 16384, 1000, 4
_BB = 512


def _onehot_body(xt_ref, o_ref):
    xt = xt_ref[...].astype(jnp.int32)
    c = jax.lax.broadcasted_iota(jnp.int32, (_P, _C, _BB), 1) + 1
    o_ref[...] = (xt[:, None, :] == c).astype(jnp.float32)


def kernel(x):
    xt = x.T
    out_t = pl.pallas_call(
        _onehot_body,
        grid=(_B // _BB,),
        in_specs=[pl.BlockSpec((_P, _BB), lambda i: (0, i))],
        out_specs=pl.BlockSpec((_P, _C, _BB), lambda i: (0, 0, i)),
        out_shape=jax.ShapeDtypeStruct((_P, _C, _B), jnp.float32),
    )(xt)
    return out_t.transpose(2, 0, 1)


# speedup vs baseline: 6.1924x; 1.0145x over previous
import jax
import jax.numpy as jnp
from jax.experimental import pallas as pl

_B, _P, _C = 16384, 1000, 4
_BB = 512


def _onehot_body(xt_ref, o_ref):
    xt = xt_ref[...]
    c = jax.lax.broadcasted_iota(jnp.int32, (1, _C, 1), 1).astype(jnp.float32) + 1.0
    o_ref[...] = (xt[:, None, :] == c).astype(jnp.float32)


def kernel(x):
    xt = x.T
    out_t = pl.pallas_call(
        _onehot_body,
        grid=(_B // _BB,),
        in_specs=[pl.BlockSpec((_P, _BB), lambda i: (0, i))],
        out_specs=pl.BlockSpec((_P, _C, _BB), lambda i: (0, 0, i)),
        out_shape=jax.ShapeDtypeStruct((_P, _C, _B), jnp.float32),
    )(xt)
    return out_t.transpose(2, 0, 1)


# speedup vs baseline: 6.4280x; 1.0380x over previous
import jax
import jax.numpy as jnp
from jax.experimental import pallas as pl

_B, _P, _C = 16384, 1000, 4
_BB = 1024


def _onehot_body(xt_ref, o_ref):
    xt = xt_ref[...]
    c = jax.lax.broadcasted_iota(jnp.int32, (1, _C, 1), 1).astype(jnp.float32) + 1.0
    o_ref[...] = (xt[:, None, :] == c).astype(jnp.float32)


def kernel(x):
    xt = x.T
    out_t = pl.pallas_call(
        _onehot_body,
        grid=(_B // _BB,),
        in_specs=[pl.BlockSpec((_P, _BB), lambda i: (0, i))],
        out_specs=pl.BlockSpec((_P, _C, _BB), lambda i: (0, 0, i)),
        out_shape=jax.ShapeDtypeStruct((_P, _C, _B), jnp.float32),
    )(xt)
    return out_t.transpose(2, 0, 1)
